# Initial kernel scaffold; baseline (speedup 1.0000x reference)
import jax
import jax.numpy as jnp
from jax.experimental import pallas as pl


def kernel(x, edge_index, W1, a_s1, a_d1, b1, W2, a_s2, a_d2, b2, Wf1, bf1, Wf2, bf2):
    raise NotImplementedError("write your pallas kernel here")


# speedup vs baseline: 40.3015x; 40.3015x over previous
import dataclasses
import functools

import jax
import jax.numpy as jnp
from jax import lax
from jax.experimental import pallas as pl
from jax.experimental.pallas import tpu as pltpu
from jax.experimental.pallas import tpu_sc as plsc

NN = 10000
NP = 10240
EE = 320000
DI = 128
DF = 128
HH = 8
CC = 16
LN = 16
NC = 2
NS = 16
BLK = 2048
CH1 = 4000
CH2 = 4000
EPT = EE // 4

_sc_mesh = plsc.VectorSubcoreMesh(
    core_axis_name="c", subcore_axis_name="s", num_cores=NC, num_subcores=NS)

_sc_params = pltpu.CompilerParams()
if "needs_layout_passes" in pltpu.CompilerParams.__dataclass_fields__:
    _sc_params = dataclasses.replace(_sc_params, needs_layout_passes=False)


def _proj1_body(x_ref, W_ref, as_w_ref, ad_w_ref, h_ref, as_ref, ad_ref):
    h = lax.dot_general(W_ref[...], x_ref[...], (((0,), (1,)), ((), ())),
                        preferred_element_type=jnp.float32)
    h_ref[...] = h
    hr = h.reshape(HH, CC, h.shape[-1])
    as_ref[...] = jnp.sum(hr * as_w_ref[...][:, :, None], axis=1)
    ad_ref[...] = jnp.sum(hr * ad_w_ref[...][:, :, None], axis=1)


def _proj2_body(xr_ref, b_ref, W_ref, as_w_ref, ad_w_ref, h_ref, as_ref, ad_ref):
    x = jnp.maximum(xr_ref[...] + b_ref[...], 0.0)
    h = lax.dot_general(W_ref[...], x, (((0,), (0,)), ((), ())),
                        preferred_element_type=jnp.float32)
    h_ref[...] = h
    hr = h.reshape(HH, CC, h.shape[-1])
    as_ref[...] = jnp.sum(hr * as_w_ref[...][:, :, None], axis=1)
    ad_ref[...] = jnp.sum(hr * ad_w_ref[...][:, :, None], axis=1)


def _mlp_body(xr_ref, b_ref, Wf1_ref, bf1_ref, Wf2_ref, bf2_ref, out_ref):
    x = jnp.maximum(xr_ref[...] + b_ref[...], 0.0)
    hid = lax.dot_general(Wf1_ref[...], x, (((0,), (0,)), ((), ())),
                          preferred_element_type=jnp.float32)
    hid = jnp.maximum(hid + bf1_ref[...], 0.0)
    out = lax.dot_general(hid, Wf2_ref[...], (((0,), (0,)), ((), ())),
                          preferred_element_type=jnp.float32)
    out_ref[...] = out + bf2_ref[...]


def _proj1(xp, W, a_s, a_d):
    return pl.pallas_call(
        _proj1_body,
        grid=(NP // BLK,),
        in_specs=[
            pl.BlockSpec((BLK, DI), lambda i: (i, 0)),
            pl.BlockSpec((DI, DF), lambda i: (0, 0)),
            pl.BlockSpec((HH, CC), lambda i: (0, 0)),
            pl.BlockSpec((HH, CC), lambda i: (0, 0)),
        ],
        out_specs=[
            pl.BlockSpec((DF, BLK), lambda i: (0, i)),
            pl.BlockSpec((HH, BLK), lambda i: (0, i)),
            pl.BlockSpec((HH, BLK), lambda i: (0, i)),
        ],
        out_shape=[
            jax.ShapeDtypeStruct((DF, NP), jnp.float32),
            jax.ShapeDtypeStruct((HH, NP), jnp.float32),
            jax.ShapeDtypeStruct((HH, NP), jnp.float32),
        ],
    )(xp, W, a_s, a_d)


def _proj2(xraw, b_col, W, a_s, a_d):
    return pl.pallas_call(
        _proj2_body,
        grid=(NP // BLK,),
        in_specs=[
            pl.BlockSpec((DF, BLK), lambda i: (0, i)),
            pl.BlockSpec((DF, 1), lambda i: (0, 0)),
            pl.BlockSpec((DF, DF), lambda i: (0, 0)),
            pl.BlockSpec((HH, CC), lambda i: (0, 0)),
            pl.BlockSpec((HH, CC), lambda i: (0, 0)),
        ],
        out_specs=[
            pl.BlockSpec((DF, BLK), lambda i: (0, i)),
            pl.BlockSpec((HH, BLK), lambda i: (0, i)),
            pl.BlockSpec((HH, BLK), lambda i: (0, i)),
        ],
        out_shape=[
            jax.ShapeDtypeStruct((DF, NP), jnp.float32),
            jax.ShapeDtypeStruct((HH, NP), jnp.float32),
            jax.ShapeDtypeStruct((HH, NP), jnp.float32),
        ],
    )(xraw, b_col, W, a_s, a_d)


def _mlp(xraw, b_col, Wf1, bf1_col, Wf2, bf2_row):
    return pl.pallas_call(
        _mlp_body,
        grid=(NP // BLK,),
        in_specs=[
            pl.BlockSpec((DF, BLK), lambda i: (0, i)),
            pl.BlockSpec((DF, 1), lambda i: (0, 0)),
            pl.BlockSpec((DF, 16), lambda i: (0, 0)),
            pl.BlockSpec((16, 1), lambda i: (0, 0)),
            pl.BlockSpec((16, DF), lambda i: (0, 0)),
            pl.BlockSpec((1, DF), lambda i: (0, 0)),
        ],
        out_specs=pl.BlockSpec((BLK, DF), lambda i: (i, 0)),
        out_shape=jax.ShapeDtypeStruct((NP, DF), jnp.float32),
    )(xraw, b_col, Wf1, bf1_col, Wf2, bf2_row)


def _p1_body(src_hbm, dst_hbm, as_hbm, ad_hbm, p_hbm, dp_hbm,
             as_col, ad_col, den, src_b, dst_b, p_b):
    c = lax.axis_index("c")
    s = lax.axis_index("s")
    h = s % HH
    q = s // HH
    part = c * 2 + q
    ebase = c * (EE // 2) + q * (EE // 4)

    pltpu.sync_copy(as_hbm.at[pl.ds(h * NP, NP)], as_col)
    pltpu.sync_copy(ad_hbm.at[pl.ds(h * NP, NP)], ad_col)

    zeros = jnp.zeros((LN,), jnp.float32)

    @pl.loop(0, NP // LN)
    def _(i):
        den[pl.ds(i * LN, LN)] = zeros

    @pl.loop(0, EPT // CH1)
    def _(ci):
        base = ebase + ci * CH1
        pltpu.sync_copy(src_hbm.at[pl.ds(base, CH1)], src_b)
        pltpu.sync_copy(dst_hbm.at[pl.ds(base, CH1)], dst_b)

        @pl.loop(0, CH1 // LN)
        def _(vi):
            sl = pl.ds(vi * LN, LN)
            sv = src_b[sl]
            dv = dst_b[sl]
            e = plsc.load_gather(as_col, [sv]) + plsc.load_gather(ad_col, [dv])
            e = jnp.maximum(e, 0.2 * e)
            pe = jnp.exp(e)
            p_b[sl] = pe
            plsc.addupdate_scatter(den, [dv], pe)

        pltpu.sync_copy(p_b, p_hbm.at[pl.ds(h * EE + base, CH1)])

    pltpu.sync_copy(den, dp_hbm.at[pl.ds((part * HH + h) * NP, NP)])


def _sc_phase1(src, dst, as_, ad_):
    kern = pl.kernel(
        _p1_body,
        out_type=(jax.ShapeDtypeStruct((HH * EE,), jnp.float32),
                  jax.ShapeDtypeStruct((4 * HH * NP,), jnp.float32)),
        mesh=_sc_mesh,
        scratch_types=[
            pltpu.VMEM((NP,), jnp.float32),
            pltpu.VMEM((NP,), jnp.float32),
            pltpu.VMEM((NP,), jnp.float32),
            pltpu.VMEM((CH1,), jnp.int32),
            pltpu.VMEM((CH1,), jnp.int32),
            pltpu.VMEM((CH1,), jnp.float32),
        ],
        compiler_params=_sc_params,
    )
    return kern(src, dst, as_, ad_)


def _p2_body(src_hbm, dst_hbm, p_hbm, dp_hbm, h_hbm, acc_hbm,
             rden, dtmp, hc0, hc1, hc2, hc3, ac0, ac1, ac2, ac3,
             src_b, dst_b, p_b):
    c = lax.axis_index("c")
    s = lax.axis_index("s")
    wid = c * NS + s
    fb = wid * 4
    h = wid // 4

    hcols = (hc0, hc1, hc2, hc3)
    accs = (ac0, ac1, ac2, ac3)

    pltpu.sync_copy(dp_hbm.at[pl.ds(h * NP, NP)], rden)
    for k in (1, 2, 3):
        pltpu.sync_copy(dp_hbm.at[pl.ds((k * HH + h) * NP, NP)], dtmp)

        @pl.loop(0, NP // LN)
        def _(i):
            sl = pl.ds(i * LN, LN)
            rden[sl] = rden[sl] + dtmp[sl]

    @pl.loop(0, NP // LN)
    def _(i):
        sl = pl.ds(i * LN, LN)
        rden[sl] = 1.0 / (rden[sl] + 1e-16)

    for r in range(4):
        pltpu.sync_copy(h_hbm.at[pl.ds((fb + r) * NP, NP)], hcols[r])

    zeros = jnp.zeros((LN,), jnp.float32)

    @pl.loop(0, NP // LN)
    def _(i):
        sl = pl.ds(i * LN, LN)
        for r in range(4):
            accs[r][sl] = zeros

    @pl.loop(0, EE // CH2)
    def _(ci):
        base = ci * CH2
        pltpu.sync_copy(src_hbm.at[pl.ds(base, CH2)], src_b)
        pltpu.sync_copy(dst_hbm.at[pl.ds(base, CH2)], dst_b)
        pltpu.sync_copy(p_hbm.at[pl.ds(h * EE + base, CH2)], p_b)

        @pl.loop(0, CH2 // LN)
        def _(vi):
            sl = pl.ds(vi * LN, LN)
            sv = src_b[sl]
            dv = dst_b[sl]
            alpha = p_b[sl] * plsc.load_gather(rden, [dv])
            for r in range(4):
                plsc.addupdate_scatter(
                    accs[r], [dv], plsc.load_gather(hcols[r], [sv]) * alpha)

    for r in range(4):
        pltpu.sync_copy(accs[r], acc_hbm.at[pl.ds((fb + r) * NP, NP)])


def _sc_phase2(src, dst, p, dp, h_t):
    kern = pl.kernel(
        _p2_body,
        out_type=jax.ShapeDtypeStruct((DF * NP,), jnp.float32),
        mesh=_sc_mesh,
        scratch_types=[
            pltpu.VMEM((NP,), jnp.float32),
            pltpu.VMEM((NP,), jnp.float32),
            pltpu.VMEM((NP,), jnp.float32),
            pltpu.VMEM((NP,), jnp.float32),
            pltpu.VMEM((NP,), jnp.float32),
            pltpu.VMEM((NP,), jnp.float32),
            pltpu.VMEM((NP,), jnp.float32),
            pltpu.VMEM((NP,), jnp.float32),
            pltpu.VMEM((NP,), jnp.float32),
            pltpu.VMEM((NP,), jnp.float32),
            pltpu.VMEM((CH2,), jnp.int32),
            pltpu.VMEM((CH2,), jnp.int32),
            pltpu.VMEM((CH2,), jnp.float32),
        ],
        compiler_params=_sc_params,
    )
    return kern(src, dst, p, dp, h_t)


def kernel(x, edge_index, W1, a_s1, a_d1, b1, W2, a_s2, a_d2, b2,
           Wf1, bf1, Wf2, bf2):
    src = edge_index[0].astype(jnp.int32)
    dst = edge_index[1].astype(jnp.int32)
    xp = jnp.pad(x, ((0, NP - NN), (0, 0)))

    h1, as1, ad1 = _proj1(xp, W1, a_s1, a_d1)
    p1, dp1 = _sc_phase1(src, dst, as1.ravel(), ad1.ravel())
    x1raw = _sc_phase2(src, dst, p1, dp1, h1.ravel())

    h2, as2, ad2 = _proj2(x1raw.reshape(DF, NP), b1.reshape(DF, 1),
                          W2, a_s2, a_d2)
    p2, dp2 = _sc_phase1(src, dst, as2.ravel(), ad2.ravel())
    x2raw = _sc_phase2(src, dst, p2, dp2, h2.ravel())

    out = _mlp(x2raw.reshape(DF, NP), b2.reshape(DF, 1), Wf1,
               bf1.reshape(16, 1), Wf2, bf2.reshape(1, DF))
    return out[:NN]


# speedup vs baseline: 49.7741x; 1.2350x over previous
import dataclasses
import functools

import jax
import jax.numpy as jnp
from jax import lax
from jax.experimental import pallas as pl
from jax.experimental.pallas import tpu as pltpu
from jax.experimental.pallas import tpu_sc as plsc

NN = 10000
NP = 10240
EE = 320000
DI = 128
DF = 128
HH = 8
CC = 16
LN = 16
NC = 2
NS = 16
BLK = 2048
CH1 = 4000
CH2 = 3200
EPT = EE // 4

_sc_mesh = plsc.VectorSubcoreMesh(
    core_axis_name="c", subcore_axis_name="s", num_cores=NC, num_subcores=NS)

_sc_params = pltpu.CompilerParams()
if "needs_layout_passes" in pltpu.CompilerParams.__dataclass_fields__:
    _sc_params = dataclasses.replace(_sc_params, needs_layout_passes=False)


def _proj1_body(x_ref, W_ref, as_w_ref, ad_w_ref, h_ref, as_ref, ad_ref):
    h = lax.dot_general(W_ref[...], x_ref[...], (((0,), (1,)), ((), ())),
                        preferred_element_type=jnp.float32)
    h_ref[...] = h
    hr = h.reshape(HH, CC, h.shape[-1])
    as_ref[...] = jnp.sum(hr * as_w_ref[...][:, :, None], axis=1)
    ad_ref[...] = jnp.sum(hr * ad_w_ref[...][:, :, None], axis=1)


def _proj2_body(xr_ref, b_ref, W_ref, as_w_ref, ad_w_ref, h_ref, as_ref, ad_ref):
    x = jnp.maximum(xr_ref[...] + b_ref[...], 0.0)
    h = lax.dot_general(W_ref[...], x, (((0,), (0,)), ((), ())),
                        preferred_element_type=jnp.float32)
    h_ref[...] = h
    hr = h.reshape(HH, CC, h.shape[-1])
    as_ref[...] = jnp.sum(hr * as_w_ref[...][:, :, None], axis=1)
    ad_ref[...] = jnp.sum(hr * ad_w_ref[...][:, :, None], axis=1)


def _mlp_body(xr_ref, b_ref, Wf1_ref, bf1_ref, Wf2_ref, bf2_ref, out_ref):
    x = jnp.maximum(xr_ref[...] + b_ref[...], 0.0)
    hid = lax.dot_general(Wf1_ref[...], x, (((0,), (0,)), ((), ())),
                          preferred_element_type=jnp.float32)
    hid = jnp.maximum(hid + bf1_ref[...], 0.0)
    out = lax.dot_general(hid, Wf2_ref[...], (((0,), (0,)), ((), ())),
                          preferred_element_type=jnp.float32)
    out_ref[...] = out + bf2_ref[...]


def _proj1(xp, W, a_s, a_d):
    return pl.pallas_call(
        _proj1_body,
        grid=(NP // BLK,),
        in_specs=[
            pl.BlockSpec((BLK, DI), lambda i: (i, 0)),
            pl.BlockSpec((DI, DF), lambda i: (0, 0)),
            pl.BlockSpec((HH, CC), lambda i: (0, 0)),
            pl.BlockSpec((HH, CC), lambda i: (0, 0)),
        ],
        out_specs=[
            pl.BlockSpec((DF, BLK), lambda i: (0, i)),
            pl.BlockSpec((HH, BLK), lambda i: (0, i)),
            pl.BlockSpec((HH, BLK), lambda i: (0, i)),
        ],
        out_shape=[
            jax.ShapeDtypeStruct((DF, NP), jnp.float32),
            jax.ShapeDtypeStruct((HH, NP), jnp.float32),
            jax.ShapeDtypeStruct((HH, NP), jnp.float32),
        ],
    )(xp, W, a_s, a_d)


def _proj2(xraw, b_col, W, a_s, a_d):
    return pl.pallas_call(
        _proj2_body,
        grid=(NP // BLK,),
        in_specs=[
            pl.BlockSpec((DF, BLK), lambda i: (0, i)),
            pl.BlockSpec((DF, 1), lambda i: (0, 0)),
            pl.BlockSpec((DF, DF), lambda i: (0, 0)),
            pl.BlockSpec((HH, CC), lambda i: (0, 0)),
            pl.BlockSpec((HH, CC), lambda i: (0, 0)),
        ],
        out_specs=[
            pl.BlockSpec((DF, BLK), lambda i: (0, i)),
            pl.BlockSpec((HH, BLK), lambda i: (0, i)),
            pl.BlockSpec((HH, BLK), lambda i: (0, i)),
        ],
        out_shape=[
            jax.ShapeDtypeStruct((DF, NP), jnp.float32),
            jax.ShapeDtypeStruct((HH, NP), jnp.float32),
            jax.ShapeDtypeStruct((HH, NP), jnp.float32),
        ],
    )(xraw, b_col, W, a_s, a_d)


def _mlp(xraw, b_col, Wf1, bf1_col, Wf2, bf2_row):
    return pl.pallas_call(
        _mlp_body,
        grid=(NP // BLK,),
        in_specs=[
            pl.BlockSpec((DF, BLK), lambda i: (0, i)),
            pl.BlockSpec((DF, 1), lambda i: (0, 0)),
            pl.BlockSpec((DF, 16), lambda i: (0, 0)),
            pl.BlockSpec((16, 1), lambda i: (0, 0)),
            pl.BlockSpec((16, DF), lambda i: (0, 0)),
            pl.BlockSpec((1, DF), lambda i: (0, 0)),
        ],
        out_specs=pl.BlockSpec((BLK, DF), lambda i: (i, 0)),
        out_shape=jax.ShapeDtypeStruct((NP, DF), jnp.float32),
    )(xraw, b_col, Wf1, bf1_col, Wf2, bf2_row)


def _p1_body(src_hbm, dst_hbm, as_hbm, ad_hbm, p_hbm, dp_hbm,
             as_col, ad_col, den,
             s0, d0, pw0, s1, d1, pw1,
             semi0, semi1, sems0, sems1):
    c = lax.axis_index("c")
    s = lax.axis_index("s")
    h = s % HH
    q = s // HH
    part = c * 2 + q
    ebase = c * (EE // 2) + q * (EE // 4)
    nch = EPT // CH1

    def issue_in(ci, sb, db, sem):
        base = ebase + ci * CH1
        pltpu.async_copy(src_hbm.at[pl.ds(base, CH1)], sb, sem)
        pltpu.async_copy(dst_hbm.at[pl.ds(base, CH1)], db, sem)

    def wait_in(sb, db, sem):
        pltpu.make_async_copy(src_hbm.at[pl.ds(0, CH1)], sb, sem).wait()
        pltpu.make_async_copy(dst_hbm.at[pl.ds(0, CH1)], db, sem).wait()

    def process(sb, db, pb):
        @pl.loop(0, CH1 // LN, unroll=5)
        def _(vi):
            sl = pl.ds(vi * LN, LN)
            sv = sb[sl]
            dv = db[sl]
            e = plsc.load_gather(as_col, [sv]) + plsc.load_gather(ad_col, [dv])
            e = jnp.maximum(e, 0.2 * e)
            pe = jnp.exp(e)
            pb[sl] = pe
            plsc.addupdate_scatter(den, [dv], pe)

    pltpu.sync_copy(as_hbm.at[pl.ds(h * NP, NP)], as_col)
    pltpu.sync_copy(ad_hbm.at[pl.ds(h * NP, NP)], ad_col)

    zeros = jnp.zeros((LN,), jnp.float32)

    @pl.loop(0, NP // LN, unroll=8)
    def _(i):
        den[pl.ds(i * LN, LN)] = zeros

    issue_in(0, s0, d0, semi0)
    issue_in(1, s1, d1, semi1)

    @pl.loop(0, nch // 2)
    def _(ci2):
        ci = 2 * ci2

        wait_in(s0, d0, semi0)

        @pl.when(ci2 > 0)
        def _():
            pltpu.make_async_copy(pw0, p_hbm.at[pl.ds(0, CH1)], sems0).wait()

        process(s0, d0, pw0)
        pltpu.async_copy(pw0, p_hbm.at[pl.ds(h * EE + ebase + ci * CH1, CH1)],
                         sems0)
        issue_in(lax.rem(ci + 2, nch), s0, d0, semi0)

        wait_in(s1, d1, semi1)

        @pl.when(ci2 > 0)
        def _():
            pltpu.make_async_copy(pw1, p_hbm.at[pl.ds(0, CH1)], sems1).wait()

        process(s1, d1, pw1)
        pltpu.async_copy(pw1,
                         p_hbm.at[pl.ds(h * EE + ebase + (ci + 1) * CH1, CH1)],
                         sems1)
        issue_in(lax.rem(ci + 3, nch), s1, d1, semi1)

    wait_in(s0, d0, semi0)
    wait_in(s1, d1, semi1)
    pltpu.make_async_copy(pw0, p_hbm.at[pl.ds(0, CH1)], sems0).wait()
    pltpu.make_async_copy(pw1, p_hbm.at[pl.ds(0, CH1)], sems1).wait()

    pltpu.sync_copy(den, dp_hbm.at[pl.ds((part * HH + h) * NP, NP)])


def _sc_phase1(src, dst, as_, ad_):
    kern = pl.kernel(
        _p1_body,
        out_type=(jax.ShapeDtypeStruct((HH * EE,), jnp.float32),
                  jax.ShapeDtypeStruct((4 * HH * NP,), jnp.float32)),
        mesh=_sc_mesh,
        scratch_types=[
            pltpu.VMEM((NP,), jnp.float32),
            pltpu.VMEM((NP,), jnp.float32),
            pltpu.VMEM((NP,), jnp.float32),
            pltpu.VMEM((CH1,), jnp.int32),
            pltpu.VMEM((CH1,), jnp.int32),
            pltpu.VMEM((CH1,), jnp.float32),
            pltpu.VMEM((CH1,), jnp.int32),
            pltpu.VMEM((CH1,), jnp.int32),
            pltpu.VMEM((CH1,), jnp.float32),
            pltpu.SemaphoreType.DMA,
            pltpu.SemaphoreType.DMA,
            pltpu.SemaphoreType.DMA,
            pltpu.SemaphoreType.DMA,
        ],
        compiler_params=_sc_params,
    )
    return kern(src, dst, as_, ad_)


def _p2_body(src_hbm, dst_hbm, p_hbm, dp_hbm, h_hbm, acc_hbm,
             rden, dtmp, hc0, hc1, hc2, hc3, ac0, ac1, ac2, ac3,
             src_b, dst_b, p_b, src_c, dst_c, p_c, semA, semB):
    c = lax.axis_index("c")
    s = lax.axis_index("s")
    wid = c * NS + s
    fb = wid * 4
    h = wid // 4

    hcols = (hc0, hc1, hc2, hc3)
    accs = (ac0, ac1, ac2, ac3)

    pltpu.sync_copy(dp_hbm.at[pl.ds(h * NP, NP)], rden)
    for k in (1, 2, 3):
        pltpu.sync_copy(dp_hbm.at[pl.ds((k * HH + h) * NP, NP)], dtmp)

        @pl.loop(0, NP // LN, unroll=8)
        def _(i):
            sl = pl.ds(i * LN, LN)
            rden[sl] = rden[sl] + dtmp[sl]

    @pl.loop(0, NP // LN, unroll=8)
    def _(i):
        sl = pl.ds(i * LN, LN)
        rden[sl] = 1.0 / (rden[sl] + 1e-16)

    for r in range(4):
        pltpu.sync_copy(h_hbm.at[pl.ds((fb + r) * NP, NP)], hcols[r])

    zeros = jnp.zeros((LN,), jnp.float32)

    @pl.loop(0, NP // LN, unroll=4)
    def _(i):
        sl = pl.ds(i * LN, LN)
        for r in range(4):
            accs[r][sl] = zeros

    nch = EE // CH2

    def issue(ci, sb, db, pb, sem):
        base = ci * CH2
        pltpu.async_copy(src_hbm.at[pl.ds(base, CH2)], sb, sem)
        pltpu.async_copy(dst_hbm.at[pl.ds(base, CH2)], db, sem)
        pltpu.async_copy(p_hbm.at[pl.ds(h * EE + base, CH2)], pb, sem)

    def wait(sb, db, pb, sem):
        pltpu.make_async_copy(src_hbm.at[pl.ds(0, CH2)], sb, sem).wait()
        pltpu.make_async_copy(dst_hbm.at[pl.ds(0, CH2)], db, sem).wait()
        pltpu.make_async_copy(p_hbm.at[pl.ds(0, CH2)], pb, sem).wait()

    def process(sb, db, pb):
        @pl.loop(0, CH2 // LN, unroll=8)
        def _(vi):
            sl = pl.ds(vi * LN, LN)
            sv = sb[sl]
            dv = db[sl]
            alpha = pb[sl] * plsc.load_gather(rden, [dv])
            for r in range(4):
                plsc.addupdate_scatter(
                    accs[r], [dv], plsc.load_gather(hcols[r], [sv]) * alpha)

    issue(0, src_b, dst_b, p_b, semA)
    issue(1, src_c, dst_c, p_c, semB)

    @pl.loop(0, nch // 2)
    def _(ci2):
        ci = 2 * ci2

        wait(src_b, dst_b, p_b, semA)
        process(src_b, dst_b, p_b)
        issue(lax.rem(ci + 2, nch), src_b, dst_b, p_b, semA)

        wait(src_c, dst_c, p_c, semB)
        process(src_c, dst_c, p_c)
        issue(lax.rem(ci + 3, nch), src_c, dst_c, p_c, semB)

    wait(src_b, dst_b, p_b, semA)
    wait(src_c, dst_c, p_c, semB)

    for r in range(4):
        pltpu.sync_copy(accs[r], acc_hbm.at[pl.ds((fb + r) * NP, NP)])


def _sc_phase2(src, dst, p, dp, h_t):
    kern = pl.kernel(
        _p2_body,
        out_type=jax.ShapeDtypeStruct((DF * NP,), jnp.float32),
        mesh=_sc_mesh,
        scratch_types=[
            pltpu.VMEM((NP,), jnp.float32),
            pltpu.VMEM((NP,), jnp.float32),
            pltpu.VMEM((NP,), jnp.float32),
            pltpu.VMEM((NP,), jnp.float32),
            pltpu.VMEM((NP,), jnp.float32),
            pltpu.VMEM((NP,), jnp.float32),
            pltpu.VMEM((NP,), jnp.float32),
            pltpu.VMEM((NP,), jnp.float32),
            pltpu.VMEM((NP,), jnp.float32),
            pltpu.VMEM((NP,), jnp.float32),
            pltpu.VMEM((CH2,), jnp.int32),
            pltpu.VMEM((CH2,), jnp.int32),
            pltpu.VMEM((CH2,), jnp.float32),
            pltpu.VMEM((CH2,), jnp.int32),
            pltpu.VMEM((CH2,), jnp.int32),
            pltpu.VMEM((CH2,), jnp.float32),
            pltpu.SemaphoreType.DMA,
            pltpu.SemaphoreType.DMA,
        ],
        compiler_params=_sc_params,
    )
    return kern(src, dst, p, dp, h_t)


def kernel(x, edge_index, W1, a_s1, a_d1, b1, W2, a_s2, a_d2, b2,
           Wf1, bf1, Wf2, bf2):
    src = edge_index[0].astype(jnp.int32)
    dst = edge_index[1].astype(jnp.int32)
    xp = jnp.pad(x, ((0, NP - NN), (0, 0)))

    h1, as1, ad1 = _proj1(xp, W1, a_s1, a_d1)
    p1, dp1 = _sc_phase1(src, dst, as1.ravel(), ad1.ravel())
    x1raw = _sc_phase2(src, dst, p1, dp1, h1.ravel())

    h2, as2, ad2 = _proj2(x1raw.reshape(DF, NP), b1.reshape(DF, 1),
                          W2, a_s2, a_d2)
    p2, dp2 = _sc_phase1(src, dst, as2.ravel(), ad2.ravel())
    x2raw = _sc_phase2(src, dst, p2, dp2, h2.ravel())

    out = _mlp(x2raw.reshape(DF, NP), b2.reshape(DF, 1), Wf1,
               bf1.reshape(16, 1), Wf2, bf2.reshape(1, DF))
    return out[:NN]


# speedup vs baseline: 120.2525x; 2.4160x over previous
import dataclasses
import functools

import jax
import jax.numpy as jnp
from jax import lax
from jax.experimental import pallas as pl
from jax.experimental.pallas import tpu as pltpu
from jax.experimental.pallas import tpu_sc as plsc

NN = 10000
NP = 10240
EE = 320000
DI = 128
DF = 128
HH = 8
CC = 16
LN = 16
NC = 2
NS = 16
BLK = 2048
CH1 = 4000
CH2 = 3200
EPT = EE // 4

_sc_mesh = plsc.VectorSubcoreMesh(
    core_axis_name="c", subcore_axis_name="s", num_cores=NC, num_subcores=NS)

_sc_params = pltpu.CompilerParams()
if "needs_layout_passes" in pltpu.CompilerParams.__dataclass_fields__:
    _sc_params = dataclasses.replace(_sc_params, needs_layout_passes=False)


def _proj1_body(x_ref, W_ref, as_w_ref, ad_w_ref, h_ref, as_ref, ad_ref):
    h = lax.dot_general(W_ref[...], x_ref[...], (((0,), (1,)), ((), ())),
                        preferred_element_type=jnp.float32)
    h_ref[...] = h
    hr = h.reshape(HH, CC, h.shape[-1])
    as_ref[...] = jnp.sum(hr * as_w_ref[...][:, :, None], axis=1)
    ad_ref[...] = jnp.sum(hr * ad_w_ref[...][:, :, None], axis=1)


def _proj2_body(xr_ref, b_ref, W_ref, as_w_ref, ad_w_ref, h_ref, as_ref, ad_ref):
    x = jnp.maximum(xr_ref[...] + b_ref[...], 0.0)
    h = lax.dot_general(W_ref[...], x, (((0,), (0,)), ((), ())),
                        preferred_element_type=jnp.float32)
    h_ref[...] = h
    hr = h.reshape(HH, CC, h.shape[-1])
    as_ref[...] = jnp.sum(hr * as_w_ref[...][:, :, None], axis=1)
    ad_ref[...] = jnp.sum(hr * ad_w_ref[...][:, :, None], axis=1)


def _mlp_body(xr_ref, b_ref, Wf1_ref, bf1_ref, Wf2_ref, bf2_ref, out_ref):
    x = jnp.maximum(xr_ref[...] + b_ref[...], 0.0)
    hid = lax.dot_general(Wf1_ref[...], x, (((0,), (0,)), ((), ())),
                          preferred_element_type=jnp.float32)
    hid = jnp.maximum(hid + bf1_ref[...], 0.0)
    out = lax.dot_general(hid, Wf2_ref[...], (((0,), (0,)), ((), ())),
                          preferred_element_type=jnp.float32)
    out_ref[...] = out + bf2_ref[...]


def _proj1(xp, W, a_s, a_d):
    return pl.pallas_call(
        _proj1_body,
        grid=(NP // BLK,),
        in_specs=[
            pl.BlockSpec((BLK, DI), lambda i: (i, 0)),
            pl.BlockSpec((DI, DF), lambda i: (0, 0)),
            pl.BlockSpec((HH, CC), lambda i: (0, 0)),
            pl.BlockSpec((HH, CC), lambda i: (0, 0)),
        ],
        out_specs=[
            pl.BlockSpec((DF, BLK), lambda i: (0, i)),
            pl.BlockSpec((HH, BLK), lambda i: (0, i)),
            pl.BlockSpec((HH, BLK), lambda i: (0, i)),
        ],
        out_shape=[
            jax.ShapeDtypeStruct((DF, NP), jnp.float32),
            jax.ShapeDtypeStruct((HH, NP), jnp.float32),
            jax.ShapeDtypeStruct((HH, NP), jnp.float32),
        ],
    )(xp, W, a_s, a_d)


def _proj2(xraw, b_col, W, a_s, a_d):
    return pl.pallas_call(
        _proj2_body,
        grid=(NP // BLK,),
        in_specs=[
            pl.BlockSpec((DF, BLK), lambda i: (0, i)),
            pl.BlockSpec((DF, 1), lambda i: (0, 0)),
            pl.BlockSpec((DF, DF), lambda i: (0, 0)),
            pl.BlockSpec((HH, CC), lambda i: (0, 0)),
            pl.BlockSpec((HH, CC), lambda i: (0, 0)),
        ],
        out_specs=[
            pl.BlockSpec((DF, BLK), lambda i: (0, i)),
            pl.BlockSpec((HH, BLK), lambda i: (0, i)),
            pl.BlockSpec((HH, BLK), lambda i: (0, i)),
        ],
        out_shape=[
            jax.ShapeDtypeStruct((DF, NP), jnp.float32),
            jax.ShapeDtypeStruct((HH, NP), jnp.float32),
            jax.ShapeDtypeStruct((HH, NP), jnp.float32),
        ],
    )(xraw, b_col, W, a_s, a_d)


def _mlp(xraw, b_col, Wf1, bf1_col, Wf2, bf2_row):
    return pl.pallas_call(
        _mlp_body,
        grid=(NP // BLK,),
        in_specs=[
            pl.BlockSpec((DF, BLK), lambda i: (0, i)),
            pl.BlockSpec((DF, 1), lambda i: (0, 0)),
            pl.BlockSpec((DF, 16), lambda i: (0, 0)),
            pl.BlockSpec((16, 1), lambda i: (0, 0)),
            pl.BlockSpec((16, DF), lambda i: (0, 0)),
            pl.BlockSpec((1, DF), lambda i: (0, 0)),
        ],
        out_specs=pl.BlockSpec((BLK, DF), lambda i: (i, 0)),
        out_shape=jax.ShapeDtypeStruct((NP, DF), jnp.float32),
    )(xraw, b_col, Wf1, bf1_col, Wf2, bf2_row)


def _p1_body(src_hbm, dst_hbm, as_hbm, ad_hbm, p_hbm, dp_hbm,
             as_col, ad_col, den,
             s0, d0, pw0, s1, d1, pw1,
             semi0, semi1, sems0, sems1):
    c = lax.axis_index("c")
    s = lax.axis_index("s")
    h = s % HH
    q = s // HH
    part = c * 2 + q
    ebase = c * (EE // 2) + q * (EE // 4)
    nch = EPT // CH1

    def issue_in(ci, sb, db, sem):
        base = ebase + ci * CH1
        pltpu.async_copy(src_hbm.at[pl.ds(base, CH1)], sb, sem)
        pltpu.async_copy(dst_hbm.at[pl.ds(base, CH1)], db, sem)

    def wait_in(sb, db, sem):
        pltpu.make_async_copy(src_hbm.at[pl.ds(0, CH1)], sb, sem).wait()
        pltpu.make_async_copy(dst_hbm.at[pl.ds(0, CH1)], db, sem).wait()

    def process(sb, db, pb):
        @plsc.parallel_loop(0, CH1 // LN, unroll=5)
        def _(vi):
            sl = pl.ds(vi * LN, LN)
            sv = sb[sl]
            dv = db[sl]
            e = plsc.load_gather(as_col, [sv]) + plsc.load_gather(ad_col, [dv])
            e = jnp.maximum(e, 0.2 * e)
            pe = jnp.exp(e)
            pb[sl] = pe
            plsc.addupdate_scatter(den, [dv], pe)

    pltpu.sync_copy(as_hbm.at[pl.ds(h * NP, NP)], as_col)
    pltpu.sync_copy(ad_hbm.at[pl.ds(h * NP, NP)], ad_col)

    zeros = jnp.zeros((LN,), jnp.float32)

    @pl.loop(0, NP // LN, unroll=8)
    def _(i):
        den[pl.ds(i * LN, LN)] = zeros

    issue_in(0, s0, d0, semi0)
    issue_in(1, s1, d1, semi1)

    @pl.loop(0, nch // 2)
    def _(ci2):
        ci = 2 * ci2

        wait_in(s0, d0, semi0)

        @pl.when(ci2 > 0)
        def _():
            pltpu.make_async_copy(pw0, p_hbm.at[pl.ds(0, CH1)], sems0).wait()

        process(s0, d0, pw0)
        pltpu.async_copy(pw0, p_hbm.at[pl.ds(h * EE + ebase + ci * CH1, CH1)],
                         sems0)
        issue_in(lax.rem(ci + 2, nch), s0, d0, semi0)

        wait_in(s1, d1, semi1)

        @pl.when(ci2 > 0)
        def _():
            pltpu.make_async_copy(pw1, p_hbm.at[pl.ds(0, CH1)], sems1).wait()

        process(s1, d1, pw1)
        pltpu.async_copy(pw1,
                         p_hbm.at[pl.ds(h * EE + ebase + (ci + 1) * CH1, CH1)],
                         sems1)
        issue_in(lax.rem(ci + 3, nch), s1, d1, semi1)

    wait_in(s0, d0, semi0)
    wait_in(s1, d1, semi1)
    pltpu.make_async_copy(pw0, p_hbm.at[pl.ds(0, CH1)], sems0).wait()
    pltpu.make_async_copy(pw1, p_hbm.at[pl.ds(0, CH1)], sems1).wait()

    pltpu.sync_copy(den, dp_hbm.at[pl.ds((part * HH + h) * NP, NP)])


def _sc_phase1(src, dst, as_, ad_):
    kern = pl.kernel(
        _p1_body,
        out_type=(jax.ShapeDtypeStruct((HH * EE,), jnp.float32),
                  jax.ShapeDtypeStruct((4 * HH * NP,), jnp.float32)),
        mesh=_sc_mesh,
        scratch_types=[
            pltpu.VMEM((NP,), jnp.float32),
            pltpu.VMEM((NP,), jnp.float32),
            pltpu.VMEM((NP,), jnp.float32),
            pltpu.VMEM((CH1,), jnp.int32),
            pltpu.VMEM((CH1,), jnp.int32),
            pltpu.VMEM((CH1,), jnp.float32),
            pltpu.VMEM((CH1,), jnp.int32),
            pltpu.VMEM((CH1,), jnp.int32),
            pltpu.VMEM((CH1,), jnp.float32),
            pltpu.SemaphoreType.DMA,
            pltpu.SemaphoreType.DMA,
            pltpu.SemaphoreType.DMA,
            pltpu.SemaphoreType.DMA,
        ],
        compiler_params=_sc_params,
    )
    return kern(src, dst, as_, ad_)


def _p2_body(src_hbm, dst_hbm, p_hbm, dp_hbm, h_hbm, acc_hbm,
             rden, dtmp, hc0, hc1, hc2, hc3, ac0, ac1, ac2, ac3,
             src_b, dst_b, p_b, src_c, dst_c, p_c, semA, semB):
    c = lax.axis_index("c")
    s = lax.axis_index("s")
    wid = c * NS + s
    fb = wid * 4
    h = wid // 4

    hcols = (hc0, hc1, hc2, hc3)
    accs = (ac0, ac1, ac2, ac3)

    pltpu.sync_copy(dp_hbm.at[pl.ds(h * NP, NP)], rden)
    for k in (1, 2, 3):
        pltpu.sync_copy(dp_hbm.at[pl.ds((k * HH + h) * NP, NP)], dtmp)

        @pl.loop(0, NP // LN, unroll=8)
        def _(i):
            sl = pl.ds(i * LN, LN)
            rden[sl] = rden[sl] + dtmp[sl]

    @pl.loop(0, NP // LN, unroll=8)
    def _(i):
        sl = pl.ds(i * LN, LN)
        rden[sl] = 1.0 / (rden[sl] + 1e-16)

    for r in range(4):
        pltpu.sync_copy(h_hbm.at[pl.ds((fb + r) * NP, NP)], hcols[r])

    zeros = jnp.zeros((LN,), jnp.float32)

    @pl.loop(0, NP // LN, unroll=4)
    def _(i):
        sl = pl.ds(i * LN, LN)
        for r in range(4):
            accs[r][sl] = zeros

    nch = EE // CH2

    def issue(ci, sb, db, pb, sem):
        base = ci * CH2
        pltpu.async_copy(src_hbm.at[pl.ds(base, CH2)], sb, sem)
        pltpu.async_copy(dst_hbm.at[pl.ds(base, CH2)], db, sem)
        pltpu.async_copy(p_hbm.at[pl.ds(h * EE + base, CH2)], pb, sem)

    def wait(sb, db, pb, sem):
        pltpu.make_async_copy(src_hbm.at[pl.ds(0, CH2)], sb, sem).wait()
        pltpu.make_async_copy(dst_hbm.at[pl.ds(0, CH2)], db, sem).wait()
        pltpu.make_async_copy(p_hbm.at[pl.ds(0, CH2)], pb, sem).wait()

    def process(sb, db, pb):
        @plsc.parallel_loop(0, CH2 // LN, unroll=4)
        def _(vi):
            sl = pl.ds(vi * LN, LN)
            sv = sb[sl]
            dv = db[sl]
            g = [plsc.load_gather(hcols[r], [sv]) for r in range(4)]
            alpha = pb[sl] * plsc.load_gather(rden, [dv])
            vals = [g[r] * alpha for r in range(4)]
            for r in range(4):
                plsc.addupdate_scatter(accs[r], [dv], vals[r])

    issue(0, src_b, dst_b, p_b, semA)
    issue(1, src_c, dst_c, p_c, semB)

    @pl.loop(0, nch // 2)
    def _(ci2):
        ci = 2 * ci2

        wait(src_b, dst_b, p_b, semA)
        process(src_b, dst_b, p_b)
        issue(lax.rem(ci + 2, nch), src_b, dst_b, p_b, semA)

        wait(src_c, dst_c, p_c, semB)
        process(src_c, dst_c, p_c)
        issue(lax.rem(ci + 3, nch), src_c, dst_c, p_c, semB)

    wait(src_b, dst_b, p_b, semA)
    wait(src_c, dst_c, p_c, semB)

    for r in range(4):
        pltpu.sync_copy(accs[r], acc_hbm.at[pl.ds((fb + r) * NP, NP)])


def _sc_phase2(src, dst, p, dp, h_t):
    kern = pl.kernel(
        _p2_body,
        out_type=jax.ShapeDtypeStruct((DF * NP,), jnp.float32),
        mesh=_sc_mesh,
        scratch_types=[
            pltpu.VMEM((NP,), jnp.float32),
            pltpu.VMEM((NP,), jnp.float32),
            pltpu.VMEM((NP,), jnp.float32),
            pltpu.VMEM((NP,), jnp.float32),
            pltpu.VMEM((NP,), jnp.float32),
            pltpu.VMEM((NP,), jnp.float32),
            pltpu.VMEM((NP,), jnp.float32),
            pltpu.VMEM((NP,), jnp.float32),
            pltpu.VMEM((NP,), jnp.float32),
            pltpu.VMEM((NP,), jnp.float32),
            pltpu.VMEM((CH2,), jnp.int32),
            pltpu.VMEM((CH2,), jnp.int32),
            pltpu.VMEM((CH2,), jnp.float32),
            pltpu.VMEM((CH2,), jnp.int32),
            pltpu.VMEM((CH2,), jnp.int32),
            pltpu.VMEM((CH2,), jnp.float32),
            pltpu.SemaphoreType.DMA,
            pltpu.SemaphoreType.DMA,
        ],
        compiler_params=_sc_params,
    )
    return kern(src, dst, p, dp, h_t)


def kernel(x, edge_index, W1, a_s1, a_d1, b1, W2, a_s2, a_d2, b2,
           Wf1, bf1, Wf2, bf2):
    src = edge_index[0].astype(jnp.int32)
    dst = edge_index[1].astype(jnp.int32)
    xp = jnp.pad(x, ((0, NP - NN), (0, 0)))

    h1, as1, ad1 = _proj1(xp, W1, a_s1, a_d1)
    p1, dp1 = _sc_phase1(src, dst, as1.ravel(), ad1.ravel())
    x1raw = _sc_phase2(src, dst, p1, dp1, h1.ravel())

    h2, as2, ad2 = _proj2(x1raw.reshape(DF, NP), b1.reshape(DF, 1),
                          W2, a_s2, a_d2)
    p2, dp2 = _sc_phase1(src, dst, as2.ravel(), ad2.ravel())
    x2raw = _sc_phase2(src, dst, p2, dp2, h2.ravel())

    out = _mlp(x2raw.reshape(DF, NP), b2.reshape(DF, 1), Wf1,
               bf1.reshape(16, 1), Wf2, bf2.reshape(1, DF))
    return out[:NN]
